# baseline (speedup 1.0000x reference)
import jax
import jax.numpy as jnp
from jax import lax
from jax.experimental import pallas as pl
from jax.experimental.pallas import tpu as pltpu
from jax.experimental.pallas import tpu_sc as plsc

N = 10000
E = 320000
F = 128
FH = F // 2
NCLASS = 64
NC = 2
NS = 16
EPT = E // NS
C = 80
NCH = EPT // C
WBT = 10
RPT = N // WBT
FV = FH // 16


def _spmm_body(h_hbm, srcr, dstr, adjr, out_hbm,
               src_all, dst_all, adj_all, rows, zbuf, acc_sh,
               gsem0, gsem1, ssem0, ssem1, zsem):
    c = lax.axis_index("c")
    s = lax.axis_index("s")

    pltpu.sync_copy(srcr.at[c, s], src_all)
    pltpu.sync_copy(dstr.at[s], dst_all)
    pltpu.sync_copy(adjr.at[s], adj_all)

    def g_start(i, b, sem):
        pltpu.async_copy(h_hbm.at[src_all.at[pl.ds(i * C, C)]],
                         rows.at[b], sem)

    def g_wait(b, sem):
        pltpu.make_async_copy(h_hbm.at[src_all.at[pl.ds(0, C)]],
                              rows.at[b], sem).wait()

    def _zrow(i, carry):
        for j in range(FV):
            zbuf[i, pl.ds(j * 16, 16)] = jnp.zeros((16,), jnp.float32)
        return carry
    lax.fori_loop(0, C, _zrow, 0)
    base_r = s * RPT

    @pl.when(s < WBT)
    def _zero_acc():
        nz = RPT // C
        rem = RPT % C
        for k in range(nz):
            pltpu.sync_copy(zbuf, acc_sh.at[pl.ds(base_r + k * C, C)])
        if rem:
            pltpu.sync_copy(zbuf.at[pl.ds(0, rem)],
                            acc_sh.at[pl.ds(base_r + nz * C, rem)])
    plsc.subcore_barrier()

    g_start(0, 0, gsem0)
    g_start(1, 1, gsem1)

    def compute(i, b):
        def _grp(g, carry):
            a16 = adj_all[pl.ds(i * C + g * 16, 16)]
            for el in range(16):
                aa = a16[el]
                e = g * 16 + el
                for j in range(FV):
                    sl = pl.ds(j * 16, 16)
                    rows[b, e, sl] = rows[b, e, sl] * aa
            return carry
        lax.fori_loop(0, C // 16, _grp, 0)

    def s_start(i, b, sem):
        pltpu.sync_copy(rows.at[b], acc_sh.at[dst_all.at[i]], add=True)

    def s_wait(b, sem):
        pass

    def _pair(k, carry):
        i0 = 2 * k
        g_wait(0, gsem0)
        compute(i0, 0)
        s_start(i0, 0, ssem0)
        g_wait(1, gsem1)
        compute(i0 + 1, 1)
        s_start(i0 + 1, 1, ssem1)
        s_wait(0, ssem0)
        g_start(i0 + 2, 0, gsem0)
        s_wait(1, ssem1)
        g_start(i0 + 3, 1, gsem1)
        return carry
    lax.fori_loop(0, NCH // 2 - 1, _pair, 0)
    g_wait(0, gsem0)
    compute(NCH - 2, 0)
    s_start(NCH - 2, 0, ssem0)
    g_wait(1, gsem1)
    compute(NCH - 1, 1)
    s_start(NCH - 1, 1, ssem1)
    s_wait(0, ssem0)
    s_wait(1, ssem1)

    plsc.subcore_barrier()

    @pl.when(s < WBT)
    def _writeback():
        pltpu.sync_copy(acc_sh.at[pl.ds(base_r, RPT)],
                        out_hbm.at[c, pl.ds(base_r, RPT)])


_spmm = pl.kernel(
    _spmm_body,
    out_type=jax.ShapeDtypeStruct((NC, N, FH), jnp.float32),
    mesh=plsc.VectorSubcoreMesh(core_axis_name="c", subcore_axis_name="s"),
    scratch_types=[
        pltpu.VMEM((EPT,), jnp.int32),
        pltpu.VMEM((NCH, C), jnp.int32),
        pltpu.VMEM((EPT,), jnp.float32),
        pltpu.VMEM((2, C, FH), jnp.float32),
        pltpu.VMEM((C, FH), jnp.float32),
        pltpu.VMEM_SHARED((N, FH), jnp.float32),
        pltpu.SemaphoreType.DMA,
        pltpu.SemaphoreType.DMA,
        pltpu.SemaphoreType.DMA,
        pltpu.SemaphoreType.DMA,
        pltpu.SemaphoreType.DMA,
    ],
    compiler_params=pltpu.CompilerParams(use_tc_tiling_on_sc=False),
)


_BR = 1000


def _split(y):
    return y[:, :FH], y[:, FH:]


def _lin_relu_body(x_ref, w_ref, b_ref, o_ref):
    y = jnp.dot(x_ref[...], w_ref[...], preferred_element_type=jnp.float32)
    y = jnp.maximum(y + b_ref[...], 0.0)
    o_ref[0], o_ref[1] = _split(y)


def _lin_relu(x, w, b):
    return pl.pallas_call(
        _lin_relu_body,
        grid=(N // _BR,),
        in_specs=[
            pl.BlockSpec((_BR, F), lambda i: (i, 0)),
            pl.BlockSpec((F, F), lambda i: (0, 0)),
            pl.BlockSpec((1, F), lambda i: (0, 0)),
        ],
        out_specs=pl.BlockSpec((NC, _BR, FH), lambda i: (0, i, 0)),
        out_shape=jax.ShapeDtypeStruct((NC, N, FH), jnp.float32),
    )(x, w, b.reshape(1, F))


def _lin_relu2_body(p_ref, w_ref, b_ref, o_ref):
    y = (jnp.dot(p_ref[0], w_ref[:FH], preferred_element_type=jnp.float32)
         + jnp.dot(p_ref[1], w_ref[FH:], preferred_element_type=jnp.float32))
    y = jnp.maximum(y + b_ref[...], 0.0)
    o_ref[0], o_ref[1] = _split(y)


def _lin_relu2(p, w, b):
    return pl.pallas_call(
        _lin_relu2_body,
        grid=(N // _BR,),
        in_specs=[
            pl.BlockSpec((NC, _BR, FH), lambda i: (0, i, 0)),
            pl.BlockSpec((F, F), lambda i: (0, 0)),
            pl.BlockSpec((1, F), lambda i: (0, 0)),
        ],
        out_specs=pl.BlockSpec((NC, _BR, FH), lambda i: (0, i, 0)),
        out_shape=jax.ShapeDtypeStruct((NC, N, FH), jnp.float32),
    )(p, w, b.reshape(1, F))


def _head_body(p_ref, wg_ref, bg_ref, w1_ref, b1_ref, w2_ref, b2_ref, o_ref):
    h = (jnp.dot(p_ref[0], wg_ref[:FH], preferred_element_type=jnp.float32)
         + jnp.dot(p_ref[1], wg_ref[FH:], preferred_element_type=jnp.float32))
    h = jnp.maximum(h + bg_ref[...], 0.0)
    t = jnp.dot(h, w1_ref[...], preferred_element_type=jnp.float32)
    t = jnp.maximum(t + b1_ref[...], 0.0)
    o_ref[...] = (
        jnp.dot(t, w2_ref[...], preferred_element_type=jnp.float32)
        + b2_ref[...]
    )


def _head(p, wg, bg, w1, b1, w2, b2):
    return pl.pallas_call(
        _head_body,
        grid=(N // _BR,),
        in_specs=[
            pl.BlockSpec((NC, _BR, FH), lambda i: (0, i, 0)),
            pl.BlockSpec((F, F), lambda i: (0, 0)),
            pl.BlockSpec((1, F), lambda i: (0, 0)),
            pl.BlockSpec((F, F), lambda i: (0, 0)),
            pl.BlockSpec((1, F), lambda i: (0, 0)),
            pl.BlockSpec((F, NCLASS), lambda i: (0, 0)),
            pl.BlockSpec((1, NCLASS), lambda i: (0, 0)),
        ],
        out_specs=pl.BlockSpec((_BR, NCLASS), lambda i: (i, 0)),
        out_shape=jax.ShapeDtypeStruct((N, NCLASS), jnp.float32),
    )(p, wg, bg.reshape(1, F), w1, b1.reshape(1, F), w2,
      b2.reshape(1, NCLASS))


@jax.jit
def kernel(features, edge_index, adj_values,
           W1, b1, Wg0, bg0, Wg1, bg1, Wg2, bg2, Wh1, bh1, Wh2, bh2):
    src = edge_index[1].reshape(NS, EPT)
    srcr = jnp.stack([src, src + N])
    dstr = edge_index[0].reshape(NS, NCH, C)
    adjr = adj_values.reshape(NS, EPT)

    h = _lin_relu(features, W1, b1)
    p = _spmm(h.reshape(NC * N, FH), srcr, dstr, adjr)
    h = _lin_relu2(p, Wg0, bg0)
    p = _spmm(h.reshape(NC * N, FH), srcr, dstr, adjr)
    h = _lin_relu2(p, Wg1, bg1)
    p = _spmm(h.reshape(NC * N, FH), srcr, dstr, adjr)
    out = _head(p, Wg2, bg2, Wh1, bh1, Wh2, bh2)
    return (out,)


# speedup vs baseline: 2.3507x; 2.3507x over previous
import jax
import jax.numpy as jnp
from jax import lax
from jax.experimental import pallas as pl
from jax.experimental.pallas import tpu as pltpu
from jax.experimental.pallas import tpu_sc as plsc

N = 10000
E = 320000
F = 128
FQ = F // 4
NCLASS = 64
NC = 2
NS = 16
EPT = E // NS
C = 80
NCH = EPT // C
WBT = 10
RPT = N // WBT
FV = FQ // 16


def _spmm_body(h_hbm, srcr, dstr, adjr, out_hbm,
               src_all, dst_all, adj_all, rows, zbuf, h_sh, acc_sh,
               gsem0, gsem1):
    c = lax.axis_index("c")
    s = lax.axis_index("s")

    pltpu.sync_copy(srcr.at[s], src_all)
    pltpu.sync_copy(dstr.at[s], dst_all)
    pltpu.sync_copy(adjr.at[s], adj_all)

    def _zrow(i, carry):
        for j in range(FV):
            zbuf[i, pl.ds(j * 16, 16)] = jnp.zeros((16,), jnp.float32)
        return carry
    lax.fori_loop(0, C, _zrow, 0)
    base_r = s * RPT

    def g_start(i, b, sem):
        pltpu.async_copy(h_sh.at[src_all.at[pl.ds(i * C, C)]],
                         rows.at[b], sem)

    def g_wait(b, sem):
        pltpu.make_async_copy(h_sh.at[src_all.at[pl.ds(0, C)]],
                              rows.at[b], sem).wait()

    def compute(i, b):
        def _grp(g, carry):
            a16 = adj_all[pl.ds(i * C + g * 16, 16)]
            for el in range(16):
                aa = a16[el]
                e = g * 16 + el
                for j in range(FV):
                    sl = pl.ds(j * 16, 16)
                    rows[b, e, sl] = rows[b, e, sl] * aa
            return carry
        lax.fori_loop(0, C // 16, _grp, 0)

    def scat(i, b):
        pltpu.sync_copy(rows.at[b], acc_sh.at[dst_all.at[i]], add=True)

    for half in range(2):
        @pl.when(s < WBT)
        def _stage():
            pltpu.sync_copy(h_hbm.at[c, half, pl.ds(base_r, RPT)],
                            h_sh.at[pl.ds(base_r, RPT)])
            nz = RPT // C
            rem = RPT % C
            for k in range(nz):
                pltpu.sync_copy(zbuf,
                                acc_sh.at[pl.ds(base_r + k * C, C)])
            if rem:
                pltpu.sync_copy(zbuf.at[pl.ds(0, rem)],
                                acc_sh.at[pl.ds(base_r + nz * C, rem)])
        plsc.subcore_barrier()

        g_start(0, 0, gsem0)
        g_start(1, 1, gsem1)

        def _pair(k, carry):
            i0 = 2 * k
            g_wait(0, gsem0)
            compute(i0, 0)
            scat(i0, 0)
            g_start(i0 + 2, 0, gsem0)
            g_wait(1, gsem1)
            compute(i0 + 1, 1)
            scat(i0 + 1, 1)
            g_start(i0 + 3, 1, gsem1)
            return carry
        lax.fori_loop(0, NCH // 2 - 1, _pair, 0)
        g_wait(0, gsem0)
        compute(NCH - 2, 0)
        scat(NCH - 2, 0)
        g_wait(1, gsem1)
        compute(NCH - 1, 1)
        scat(NCH - 1, 1)

        plsc.subcore_barrier()

        @pl.when(s < WBT)
        def _writeback():
            pltpu.sync_copy(acc_sh.at[pl.ds(base_r, RPT)],
                            out_hbm.at[c, half, pl.ds(base_r, RPT)])


_spmm = pl.kernel(
    _spmm_body,
    out_type=jax.ShapeDtypeStruct((NC, 2, N, FQ), jnp.float32),
    mesh=plsc.VectorSubcoreMesh(core_axis_name="c", subcore_axis_name="s"),
    scratch_types=[
        pltpu.VMEM((EPT,), jnp.int32),
        pltpu.VMEM((NCH, C), jnp.int32),
        pltpu.VMEM((EPT,), jnp.float32),
        pltpu.VMEM((2, C, FQ), jnp.float32),
        pltpu.VMEM((C, FQ), jnp.float32),
        pltpu.VMEM_SHARED((N, FQ), jnp.float32),
        pltpu.VMEM_SHARED((N, FQ), jnp.float32),
        pltpu.SemaphoreType.DMA,
        pltpu.SemaphoreType.DMA,
    ],
    compiler_params=pltpu.CompilerParams(use_tc_tiling_on_sc=False),
)


_BR = 1000


def _store_quarters(o_ref, y):
    o_ref[0, 0] = y[:, 0 * FQ:1 * FQ]
    o_ref[0, 1] = y[:, 1 * FQ:2 * FQ]
    o_ref[1, 0] = y[:, 2 * FQ:3 * FQ]
    o_ref[1, 1] = y[:, 3 * FQ:4 * FQ]


def _dot_quarters(p_ref, w_ref):
    return sum(
        jnp.dot(p_ref[q // 2, q % 2], w_ref[q * FQ:(q + 1) * FQ],
                preferred_element_type=jnp.float32)
        for q in range(4)
    )


def _lin_relu_body(x_ref, w_ref, b_ref, o_ref):
    y = jnp.dot(x_ref[...], w_ref[...], preferred_element_type=jnp.float32)
    _store_quarters(o_ref, jnp.maximum(y + b_ref[...], 0.0))


def _lin_relu(x, w, b):
    return pl.pallas_call(
        _lin_relu_body,
        grid=(N // _BR,),
        in_specs=[
            pl.BlockSpec((_BR, F), lambda i: (i, 0)),
            pl.BlockSpec((F, F), lambda i: (0, 0)),
            pl.BlockSpec((1, F), lambda i: (0, 0)),
        ],
        out_specs=pl.BlockSpec((NC, 2, _BR, FQ), lambda i: (0, 0, i, 0)),
        out_shape=jax.ShapeDtypeStruct((NC, 2, N, FQ), jnp.float32),
    )(x, w, b.reshape(1, F))


def _lin_relu2_body(p_ref, w_ref, b_ref, o_ref):
    y = _dot_quarters(p_ref, w_ref)
    _store_quarters(o_ref, jnp.maximum(y + b_ref[...], 0.0))


def _lin_relu2(p, w, b):
    return pl.pallas_call(
        _lin_relu2_body,
        grid=(N // _BR,),
        in_specs=[
            pl.BlockSpec((NC, 2, _BR, FQ), lambda i: (0, 0, i, 0)),
            pl.BlockSpec((F, F), lambda i: (0, 0)),
            pl.BlockSpec((1, F), lambda i: (0, 0)),
        ],
        out_specs=pl.BlockSpec((NC, 2, _BR, FQ), lambda i: (0, 0, i, 0)),
        out_shape=jax.ShapeDtypeStruct((NC, 2, N, FQ), jnp.float32),
    )(p, w, b.reshape(1, F))


def _head_body(h_ref, w1_ref, b1_ref, w2_ref, b2_ref, o_ref):
    t = _dot_quarters(h_ref, w1_ref)
    t = jnp.maximum(t + b1_ref[...], 0.0)
    o_ref[...] = (
        jnp.dot(t, w2_ref[...], preferred_element_type=jnp.float32)
        + b2_ref[...]
    )


def _head(h, w1, b1, w2, b2):
    return pl.pallas_call(
        _head_body,
        grid=(N // _BR,),
        in_specs=[
            pl.BlockSpec((NC, 2, _BR, FQ), lambda i: (0, 0, i, 0)),
            pl.BlockSpec((F, F), lambda i: (0, 0)),
            pl.BlockSpec((1, F), lambda i: (0, 0)),
            pl.BlockSpec((F, NCLASS), lambda i: (0, 0)),
            pl.BlockSpec((1, NCLASS), lambda i: (0, 0)),
        ],
        out_specs=pl.BlockSpec((_BR, NCLASS), lambda i: (i, 0)),
        out_shape=jax.ShapeDtypeStruct((N, NCLASS), jnp.float32),
    )(h, w1, b1.reshape(1, F), w2, b2.reshape(1, NCLASS))


@jax.jit
def kernel(features, edge_index, adj_values,
           W1, b1, Wg0, bg0, Wg1, bg1, Wg2, bg2, Wh1, bh1, Wh2, bh2):
    srcr = edge_index[1].reshape(NS, EPT)
    dstr = edge_index[0].reshape(NS, NCH, C)
    adjr = adj_values.reshape(NS, EPT)

    h = _lin_relu(features, W1, b1)

    def _layer(hc, wb):
        w, b = wb
        p = _spmm(hc, srcr, dstr, adjr)
        return _lin_relu2(p, w, b), None

    wg = jnp.stack([Wg0, Wg1, Wg2])
    bg = jnp.stack([bg0, bg1, bg2])
    h, _ = lax.scan(_layer, h, (wg, bg))
    out = _head(h, Wh1, bh1, Wh2, bh2)
    return (out,)


# speedup vs baseline: 2.5308x; 1.0766x over previous
import jax
import jax.numpy as jnp
from jax import lax
from jax.experimental import pallas as pl
from jax.experimental.pallas import tpu as pltpu
from jax.experimental.pallas import tpu_sc as plsc

N = 10000
E = 320000
F = 128
FQ = F // 4
NCLASS = 64
NC = 2
NS = 16
EPT = E // NS
C = 80
NCH = EPT // C
WBT = 10
RPT = N // WBT
FV = FQ // 16


def _spmm_body(h_hbm, srcr, dstr, adjr, out_hbm,
               src_all, dst_all, adj_all, rows, zbuf, h_sh, acc_sh,
               gs0, gs1, gs2, gs3, ss0, ss1, ss2, ss3):
    gsem = (gs0, gs1, gs2, gs3)
    ssem = (ss0, ss1, ss2, ss3)
    c = lax.axis_index("c")
    s = lax.axis_index("s")

    pltpu.sync_copy(srcr.at[s], src_all)
    pltpu.sync_copy(dstr.at[s], dst_all)
    pltpu.sync_copy(adjr.at[s], adj_all)

    def _zrow(i, carry):
        for j in range(FV):
            zbuf[i, pl.ds(j * 16, 16)] = jnp.zeros((16,), jnp.float32)
        return carry
    lax.fori_loop(0, C, _zrow, 0)
    base_r = s * RPT

    def g_start(i, b, sem):
        pltpu.async_copy(h_sh.at[src_all.at[pl.ds(i * C, C)]],
                         rows.at[b], sem)

    def g_wait(b, sem):
        pltpu.make_async_copy(h_sh.at[src_all.at[pl.ds(0, C)]],
                              rows.at[b], sem).wait()

    def compute(i, b):
        def _grp(g, carry):
            a16 = adj_all[pl.ds(i * C + g * 16, 16)]
            for el in range(16):
                aa = a16[el]
                e = g * 16 + el
                for j in range(FV):
                    sl = pl.ds(j * 16, 16)
                    rows[b, e, sl] = rows[b, e, sl] * aa
            return carry
        lax.fori_loop(0, C // 16, _grp, 0)

    def s_start(i, b, sem):
        pltpu.async_copy(rows.at[b], acc_sh.at[dst_all.at[i]], sem, add=True)

    def s_wait(b, sem):
        pltpu.make_async_copy(rows.at[b], acc_sh.at[dst_all.at[0]],
                              sem).wait()

    for half in range(2):
        @pl.when(s < WBT)
        def _stage():
            pltpu.sync_copy(h_hbm.at[c, half, pl.ds(base_r, RPT)],
                            h_sh.at[pl.ds(base_r, RPT)])
            nz = RPT // C
            rem = RPT % C
            for k in range(nz):
                pltpu.sync_copy(zbuf,
                                acc_sh.at[pl.ds(base_r + k * C, C)])
            if rem:
                pltpu.sync_copy(zbuf.at[pl.ds(0, rem)],
                                acc_sh.at[pl.ds(base_r + nz * C, rem)])
        plsc.subcore_barrier()

        g_start(0, 0, gsem[0])
        g_start(1, 1, gsem[1])
        g_wait(0, gsem[0])
        compute(0, 0)
        s_start(0, 0, ssem[0])
        g_start(2, 2, gsem[2])
        g_wait(1, gsem[1])
        compute(1, 1)
        s_start(1, 1, ssem[1])
        g_start(3, 3, gsem[3])
        g_wait(2, gsem[2])
        compute(2, 2)
        s_start(2, 2, ssem[2])
        s_wait(0, ssem[0])
        g_start(4, 0, gsem[0])
        g_wait(3, gsem[3])
        compute(3, 3)
        s_start(3, 3, ssem[3])
        s_wait(1, ssem[1])
        g_start(5, 1, gsem[1])

        def _quad(k, carry):
            i0 = 4 * k
            for b in range(4):
                g_wait(b, gsem[b])
                compute(i0 + b, b)
                s_start(i0 + b, b, ssem[b])
                bn = (b + 2) % 4
                s_wait(bn, ssem[bn])
                g_start(i0 + b + 2, bn, gsem[bn])
            return carry
        lax.fori_loop(1, NCH // 4, _quad, 0)
        g_wait(0, gsem[0])
        compute(NCH - 2, 0)
        s_start(NCH - 2, 0, ssem[0])
        g_wait(1, gsem[1])
        compute(NCH - 1, 1)
        s_start(NCH - 1, 1, ssem[1])
        s_wait(2, ssem[2])
        s_wait(3, ssem[3])
        s_wait(0, ssem[0])
        s_wait(1, ssem[1])

        plsc.subcore_barrier()

        @pl.when(s < WBT)
        def _writeback():
            pltpu.sync_copy(acc_sh.at[pl.ds(base_r, RPT)],
                            out_hbm.at[c, half, pl.ds(base_r, RPT)])


_spmm = pl.kernel(
    _spmm_body,
    out_type=jax.ShapeDtypeStruct((NC, 2, N, FQ), jnp.float32),
    mesh=plsc.VectorSubcoreMesh(core_axis_name="c", subcore_axis_name="s"),
    scratch_types=[
        pltpu.VMEM((EPT,), jnp.int32),
        pltpu.VMEM((NCH, C), jnp.int32),
        pltpu.VMEM((EPT,), jnp.float32),
        pltpu.VMEM((4, C, FQ), jnp.float32),
        pltpu.VMEM((C, FQ), jnp.float32),
        pltpu.VMEM_SHARED((N, FQ), jnp.float32),
        pltpu.VMEM_SHARED((N, FQ), jnp.float32),
        pltpu.SemaphoreType.DMA,
        pltpu.SemaphoreType.DMA,
        pltpu.SemaphoreType.DMA,
        pltpu.SemaphoreType.DMA,
        pltpu.SemaphoreType.DMA,
        pltpu.SemaphoreType.DMA,
        pltpu.SemaphoreType.DMA,
        pltpu.SemaphoreType.DMA,
    ],
    compiler_params=pltpu.CompilerParams(use_tc_tiling_on_sc=False),
)


_BR = 1000


def _store_quarters(o_ref, y):
    o_ref[0, 0] = y[:, 0 * FQ:1 * FQ]
    o_ref[0, 1] = y[:, 1 * FQ:2 * FQ]
    o_ref[1, 0] = y[:, 2 * FQ:3 * FQ]
    o_ref[1, 1] = y[:, 3 * FQ:4 * FQ]


def _dot_quarters(p_ref, w_ref):
    return sum(
        jnp.dot(p_ref[q // 2, q % 2], w_ref[q * FQ:(q + 1) * FQ],
                preferred_element_type=jnp.float32)
        for q in range(4)
    )


def _lin_relu_body(x_ref, w_ref, b_ref, o_ref):
    y = jnp.dot(x_ref[...], w_ref[...], preferred_element_type=jnp.float32)
    _store_quarters(o_ref, jnp.maximum(y + b_ref[...], 0.0))


def _lin_relu(x, w, b):
    return pl.pallas_call(
        _lin_relu_body,
        grid=(N // _BR,),
        in_specs=[
            pl.BlockSpec((_BR, F), lambda i: (i, 0)),
            pl.BlockSpec((F, F), lambda i: (0, 0)),
            pl.BlockSpec((1, F), lambda i: (0, 0)),
        ],
        out_specs=pl.BlockSpec((NC, 2, _BR, FQ), lambda i: (0, 0, i, 0)),
        out_shape=jax.ShapeDtypeStruct((NC, 2, N, FQ), jnp.float32),
    )(x, w, b.reshape(1, F))


def _lin_relu2_body(p_ref, w_ref, b_ref, o_ref):
    y = _dot_quarters(p_ref, w_ref)
    _store_quarters(o_ref, jnp.maximum(y + b_ref[...], 0.0))


def _lin_relu2(p, w, b):
    return pl.pallas_call(
        _lin_relu2_body,
        grid=(N // _BR,),
        in_specs=[
            pl.BlockSpec((NC, 2, _BR, FQ), lambda i: (0, 0, i, 0)),
            pl.BlockSpec((F, F), lambda i: (0, 0)),
            pl.BlockSpec((1, F), lambda i: (0, 0)),
        ],
        out_specs=pl.BlockSpec((NC, 2, _BR, FQ), lambda i: (0, 0, i, 0)),
        out_shape=jax.ShapeDtypeStruct((NC, 2, N, FQ), jnp.float32),
    )(p, w, b.reshape(1, F))


def _head_body(h_ref, w1_ref, b1_ref, w2_ref, b2_ref, o_ref):
    t = _dot_quarters(h_ref, w1_ref)
    t = jnp.maximum(t + b1_ref[...], 0.0)
    o_ref[...] = (
        jnp.dot(t, w2_ref[...], preferred_element_type=jnp.float32)
        + b2_ref[...]
    )


def _head(h, w1, b1, w2, b2):
    return pl.pallas_call(
        _head_body,
        grid=(N // _BR,),
        in_specs=[
            pl.BlockSpec((NC, 2, _BR, FQ), lambda i: (0, 0, i, 0)),
            pl.BlockSpec((F, F), lambda i: (0, 0)),
            pl.BlockSpec((1, F), lambda i: (0, 0)),
            pl.BlockSpec((F, NCLASS), lambda i: (0, 0)),
            pl.BlockSpec((1, NCLASS), lambda i: (0, 0)),
        ],
        out_specs=pl.BlockSpec((_BR, NCLASS), lambda i: (i, 0)),
        out_shape=jax.ShapeDtypeStruct((N, NCLASS), jnp.float32),
    )(h, w1, b1.reshape(1, F), w2, b2.reshape(1, NCLASS))


@jax.jit
def kernel(features, edge_index, adj_values,
           W1, b1, Wg0, bg0, Wg1, bg1, Wg2, bg2, Wh1, bh1, Wh2, bh2):
    srcr = edge_index[1].reshape(NS, EPT)
    dstr = edge_index[0].reshape(NS, NCH, C)
    adjr = adj_values.reshape(NS, EPT)

    h = _lin_relu(features, W1, b1)

    def _layer(hc, wb):
        w, b = wb
        p = _spmm(hc, srcr, dstr, adjr)
        return _lin_relu2(p, w, b), None

    wg = jnp.stack([Wg0, Wg1, Wg2])
    bg = jnp.stack([bg0, bg1, bg2])
    h, _ = lax.scan(_layer, h, (wg, bg))
    out = _head(h, Wh1, bh1, Wh2, bh2)
    return (out,)


# speedup vs baseline: 2.6132x; 1.0326x over previous
import jax
import jax.numpy as jnp
from jax import lax
from jax.experimental import pallas as pl
from jax.experimental.pallas import tpu as pltpu
from jax.experimental.pallas import tpu_sc as plsc

N = 10000
E = 320000
F = 128
FQ = F // 4
NCLASS = 64
NC = 2
NS = 16
EPT = E // NS
C = 80
NCH = EPT // C
WBT = 10
RPT = N // WBT
FV = FQ // 16
ZR = 200
NZ = RPT // ZR


def _spmm_body(h_hbm, srcr, dstr, adjr, out_hbm,
               src_all, dst_all, adj_all, rows, zbuf, h_sh, acc_sh,
               gs0, gs1, gs2, gs3, ss0, ss1, ss2, ss3):
    gsem = (gs0, gs1, gs2, gs3)
    ssem = (ss0, ss1, ss2, ss3)
    c = lax.axis_index("c")
    s = lax.axis_index("s")

    base_r = s * RPT
    zsem = (ss0, ss1, ss2, ss3, ss0)

    def _issue_zeros():
        for k in range(NZ):
            pltpu.async_copy(zbuf, acc_sh.at[pl.ds(base_r + k * ZR, ZR)],
                             zsem[k])

    def _wait_zeros():
        for k in range(NZ):
            pltpu.make_async_copy(zbuf,
                                  acc_sh.at[pl.ds(base_r + k * ZR, ZR)],
                                  zsem[k]).wait()

    def _issue_slab(half):
        pltpu.async_copy(h_hbm.at[c, half, pl.ds(base_r, RPT)],
                         h_sh.at[pl.ds(base_r, RPT)], gs3)

    def _wait_slab(half):
        pltpu.make_async_copy(h_hbm.at[c, half, pl.ds(base_r, RPT)],
                              h_sh.at[pl.ds(base_r, RPT)], gs3).wait()

    pltpu.async_copy(srcr.at[s], src_all, gs0)
    pltpu.async_copy(dstr.at[s], dst_all, gs1)
    pltpu.async_copy(adjr.at[s], adj_all, gs2)

    @pl.when(s < WBT)
    def _stage0():
        _issue_slab(0)

    def _zrow(i, carry):
        for j in range(FV):
            zbuf[i, pl.ds(j * 16, 16)] = jnp.zeros((16,), jnp.float32)
        return carry
    lax.fori_loop(0, ZR, _zrow, 0)

    @pl.when(s < WBT)
    def _zero0():
        _issue_zeros()

    pltpu.make_async_copy(srcr.at[s], src_all, gs0).wait()
    pltpu.make_async_copy(dstr.at[s], dst_all, gs1).wait()
    pltpu.make_async_copy(adjr.at[s], adj_all, gs2).wait()

    @pl.when(s < WBT)
    def _wait0():
        _wait_slab(0)
        _wait_zeros()

    def g_start(i, b, sem):
        pltpu.async_copy(h_sh.at[src_all.at[pl.ds(i * C, C)]],
                         rows.at[b], sem)

    def g_wait(b, sem):
        pltpu.make_async_copy(h_sh.at[src_all.at[pl.ds(0, C)]],
                              rows.at[b], sem).wait()

    def compute(i, b):
        def _grp(g, carry):
            a16 = adj_all[pl.ds(i * C + g * 16, 16)]
            for el in range(16):
                aa = a16[el]
                e = g * 16 + el
                for j in range(FV):
                    sl = pl.ds(j * 16, 16)
                    rows[b, e, sl] = rows[b, e, sl] * aa
            return carry
        lax.fori_loop(0, C // 16, _grp, 0)

    def s_start(i, b, sem):
        pltpu.async_copy(rows.at[b], acc_sh.at[dst_all.at[i]], sem, add=True)

    def s_wait(b, sem):
        pltpu.make_async_copy(rows.at[b], acc_sh.at[dst_all.at[0]],
                              sem).wait()

    for half in range(2):
        plsc.subcore_barrier()

        g_start(0, 0, gsem[0])
        g_start(1, 1, gsem[1])
        g_wait(0, gsem[0])
        compute(0, 0)
        s_start(0, 0, ssem[0])
        g_start(2, 2, gsem[2])
        g_wait(1, gsem[1])
        compute(1, 1)
        s_start(1, 1, ssem[1])
        g_start(3, 3, gsem[3])
        g_wait(2, gsem[2])
        compute(2, 2)
        s_start(2, 2, ssem[2])
        s_wait(0, ssem[0])
        g_start(4, 0, gsem[0])
        g_wait(3, gsem[3])
        compute(3, 3)
        s_start(3, 3, ssem[3])
        s_wait(1, ssem[1])
        g_start(5, 1, gsem[1])

        def _quad(k, carry):
            i0 = 4 * k
            for b in range(4):
                g_wait(b, gsem[b])
                compute(i0 + b, b)
                s_start(i0 + b, b, ssem[b])
                bn = (b + 2) % 4
                s_wait(bn, ssem[bn])
                g_start(i0 + b + 2, bn, gsem[bn])
            return carry
        lax.fori_loop(1, NCH // 4, _quad, 0)
        g_wait(0, gsem[0])
        compute(NCH - 2, 0)
        s_start(NCH - 2, 0, ssem[0])
        g_wait(1, gsem[1])
        compute(NCH - 1, 1)
        s_start(NCH - 1, 1, ssem[1])
        s_wait(2, ssem[2])
        s_wait(3, ssem[3])
        s_wait(0, ssem[0])
        s_wait(1, ssem[1])

        plsc.subcore_barrier()

        if half == 0:
            @pl.when(s < WBT)
            def _inter():
                pltpu.async_copy(acc_sh.at[pl.ds(base_r, RPT)],
                                 out_hbm.at[c, 0, pl.ds(base_r, RPT)], gs0)
                _issue_slab(1)
                pltpu.make_async_copy(
                    acc_sh.at[pl.ds(base_r, RPT)],
                    out_hbm.at[c, 0, pl.ds(base_r, RPT)], gs0).wait()
                _issue_zeros()
                _wait_slab(1)
                _wait_zeros()
        else:
            @pl.when(s < WBT)
            def _writeback():
                pltpu.sync_copy(acc_sh.at[pl.ds(base_r, RPT)],
                                out_hbm.at[c, 1, pl.ds(base_r, RPT)])


_spmm = pl.kernel(
    _spmm_body,
    out_type=jax.ShapeDtypeStruct((NC, 2, N, FQ), jnp.float32),
    mesh=plsc.VectorSubcoreMesh(core_axis_name="c", subcore_axis_name="s"),
    scratch_types=[
        pltpu.VMEM((EPT,), jnp.int32),
        pltpu.VMEM((NCH, C), jnp.int32),
        pltpu.VMEM((EPT,), jnp.float32),
        pltpu.VMEM((4, C, FQ), jnp.float32),
        pltpu.VMEM((ZR, FQ), jnp.float32),
        pltpu.VMEM_SHARED((N, FQ), jnp.float32),
        pltpu.VMEM_SHARED((N, FQ), jnp.float32),
        pltpu.SemaphoreType.DMA,
        pltpu.SemaphoreType.DMA,
        pltpu.SemaphoreType.DMA,
        pltpu.SemaphoreType.DMA,
        pltpu.SemaphoreType.DMA,
        pltpu.SemaphoreType.DMA,
        pltpu.SemaphoreType.DMA,
        pltpu.SemaphoreType.DMA,
    ],
    compiler_params=pltpu.CompilerParams(use_tc_tiling_on_sc=False),
)


_BR = 1000


def _store_quarters(o_ref, y):
    o_ref[0, 0] = y[:, 0 * FQ:1 * FQ]
    o_ref[0, 1] = y[:, 1 * FQ:2 * FQ]
    o_ref[1, 0] = y[:, 2 * FQ:3 * FQ]
    o_ref[1, 1] = y[:, 3 * FQ:4 * FQ]


def _dot_quarters(p_ref, w_ref):
    return sum(
        jnp.dot(p_ref[q // 2, q % 2], w_ref[q * FQ:(q + 1) * FQ],
                preferred_element_type=jnp.float32)
        for q in range(4)
    )


def _lin_relu_body(x_ref, w_ref, b_ref, o_ref):
    y = jnp.dot(x_ref[...], w_ref[...], preferred_element_type=jnp.float32)
    _store_quarters(o_ref, jnp.maximum(y + b_ref[...], 0.0))


def _lin_relu(x, w, b):
    return pl.pallas_call(
        _lin_relu_body,
        grid=(N // _BR,),
        in_specs=[
            pl.BlockSpec((_BR, F), lambda i: (i, 0)),
            pl.BlockSpec((F, F), lambda i: (0, 0)),
            pl.BlockSpec((1, F), lambda i: (0, 0)),
        ],
        out_specs=pl.BlockSpec((NC, 2, _BR, FQ), lambda i: (0, 0, i, 0)),
        out_shape=jax.ShapeDtypeStruct((NC, 2, N, FQ), jnp.float32),
    )(x, w, b.reshape(1, F))


def _lin_relu2_body(p_ref, w_ref, b_ref, o_ref):
    y = _dot_quarters(p_ref, w_ref)
    _store_quarters(o_ref, jnp.maximum(y + b_ref[...], 0.0))


def _lin_relu2(p, w, b):
    return pl.pallas_call(
        _lin_relu2_body,
        grid=(N // _BR,),
        in_specs=[
            pl.BlockSpec((NC, 2, _BR, FQ), lambda i: (0, 0, i, 0)),
            pl.BlockSpec((F, F), lambda i: (0, 0)),
            pl.BlockSpec((1, F), lambda i: (0, 0)),
        ],
        out_specs=pl.BlockSpec((NC, 2, _BR, FQ), lambda i: (0, 0, i, 0)),
        out_shape=jax.ShapeDtypeStruct((NC, 2, N, FQ), jnp.float32),
    )(p, w, b.reshape(1, F))


def _head_body(h_ref, w1_ref, b1_ref, w2_ref, b2_ref, o_ref):
    t = _dot_quarters(h_ref, w1_ref)
    t = jnp.maximum(t + b1_ref[...], 0.0)
    o_ref[...] = (
        jnp.dot(t, w2_ref[...], preferred_element_type=jnp.float32)
        + b2_ref[...]
    )


def _head(h, w1, b1, w2, b2):
    return pl.pallas_call(
        _head_body,
        grid=(N // _BR,),
        in_specs=[
            pl.BlockSpec((NC, 2, _BR, FQ), lambda i: (0, 0, i, 0)),
            pl.BlockSpec((F, F), lambda i: (0, 0)),
            pl.BlockSpec((1, F), lambda i: (0, 0)),
            pl.BlockSpec((F, NCLASS), lambda i: (0, 0)),
            pl.BlockSpec((1, NCLASS), lambda i: (0, 0)),
        ],
        out_specs=pl.BlockSpec((_BR, NCLASS), lambda i: (i, 0)),
        out_shape=jax.ShapeDtypeStruct((N, NCLASS), jnp.float32),
    )(h, w1, b1.reshape(1, F), w2, b2.reshape(1, NCLASS))


@jax.jit
def kernel(features, edge_index, adj_values,
           W1, b1, Wg0, bg0, Wg1, bg1, Wg2, bg2, Wh1, bh1, Wh2, bh2):
    srcr = edge_index[1].reshape(NS, EPT)
    dstr = edge_index[0].reshape(NS, NCH, C)
    adjr = adj_values.reshape(NS, EPT)

    h = _lin_relu(features, W1, b1)

    def _layer(hc, wb):
        w, b = wb
        p = _spmm(hc, srcr, dstr, adjr)
        return _lin_relu2(p, w, b), None

    wg = jnp.stack([Wg0, Wg1, Wg2])
    bg = jnp.stack([bg0, bg1, bg2])
    h, _ = lax.scan(_layer, h, (wg, bg))
    out = _head(h, Wh1, bh1, Wh2, bh2)
    return (out,)
